# Initial kernel scaffold; baseline (speedup 1.0000x reference)
import jax
import jax.numpy as jnp
from jax.experimental import pallas as pl


def kernel(x, edge_index, batch, W1, b1, W2, b2, Wc, bc):
    raise NotImplementedError("write your pallas kernel here")


# speedup vs baseline: 8.8111x; 8.8111x over previous
import functools

import jax
import jax.numpy as jnp
from jax import lax
from jax.experimental import pallas as pl
from jax.experimental.pallas import tpu as pltpu
from jax.experimental.pallas import tpu_sc as plsc

N_NODES = 10000
D_FEAT = 128
HIDDEN = 32
N_GRAPHS = 64
N_EDGES = 320000

NC = 2
NS = 16
NW = NC * NS
LANES = 16

BATCH_SZ = 128
NB = 80
EPW = NB * BATCH_SZ
PAD_E = NW * EPW
ROWS_PAD = 10112
RPS = ROWS_PAD // NS
DUMMY_ROW = N_NODES


def _xw_body(x_ref, w_ref, o_ref):
    o_ref[...] = jnp.dot(x_ref[...], w_ref[...], preferred_element_type=jnp.float32)


def _tc_xw(x, W1):
    return pl.pallas_call(
        _xw_body,
        out_shape=jax.ShapeDtypeStruct((N_NODES, HIDDEN), jnp.float32),
    )(x, W1)


def _sc_scatter(xa, srcm, dstm):
    mesh = plsc.VectorSubcoreMesh(
        core_axis_name="c", subcore_axis_name="s", num_cores=NC, num_subcores=NS
    )

    @functools.partial(
        pl.kernel,
        out_type=jax.ShapeDtypeStruct((NC, ROWS_PAD, HIDDEN), jnp.float32),
        mesh=mesh,
        scratch_types=[
            pltpu.VMEM((NB, BATCH_SZ), jnp.int32),
            pltpu.VMEM((NB, BATCH_SZ), jnp.int32),
            pltpu.VMEM((BATCH_SZ, HIDDEN), jnp.float32),
            pltpu.VMEM((RPS, HIDDEN), jnp.float32),
            pltpu.VMEM_SHARED((ROWS_PAD, HIDDEN), jnp.float32),
            pltpu.SemaphoreType.DMA,
        ],
        compiler_params=pltpu.CompilerParams(use_tc_tiling_on_sc=False),
    )
    def k(xa_hbm, srcm_hbm, dstm_hbm, out_hbm, srcbuf, dstbuf, rows, zbuf, aggsh, sem):
        c = lax.axis_index("c")
        s = lax.axis_index("s")
        wid = s * NC + c

        def zrow(r, carry):
            z = jnp.zeros((LANES,), jnp.float32)
            zbuf[r, pl.ds(0, LANES)] = z
            zbuf[r, pl.ds(LANES, LANES)] = z
            return carry

        lax.fori_loop(0, RPS, zrow, 0)
        pltpu.sync_copy(zbuf, aggsh.at[pl.ds(s * RPS, RPS)])

        pltpu.sync_copy(srcm_hbm.at[pl.ds(wid * NB, NB)], srcbuf)
        pltpu.sync_copy(dstm_hbm.at[pl.ds(wid * NB, NB)], dstbuf)
        plsc.subcore_barrier()

        def body(j, carry):
            pltpu.async_copy(xa_hbm.at[srcbuf.at[j]], rows, sem).wait()
            pltpu.sync_copy(rows, aggsh.at[dstbuf.at[j]], add=True)
            return carry

        lax.fori_loop(0, NB, body, 0)
        plsc.subcore_barrier()

        pltpu.sync_copy(
            aggsh.at[pl.ds(s * RPS, RPS)], out_hbm.at[c, pl.ds(s * RPS, RPS)]
        )

    return k(xa, srcm, dstm)


def _tail_body(xa_ref, p0_ref, p1_ref, bt_ref, W2_ref, b2_ref, Wc_ref, bc_ref,
               b1_ref, o_ref):
    h1 = jnp.maximum(xa_ref[...] + p0_ref[...] + p1_ref[...] + b1_ref[...], 0.0)
    h = jnp.dot(h1, W2_ref[...], preferred_element_type=jnp.float32) + b2_ref[...]
    h = jnp.maximum(h, 0.0)
    gids = lax.broadcasted_iota(jnp.int32, (N_GRAPHS, N_NODES), 0)
    onehot_t = (gids == bt_ref[...]).astype(jnp.float32)
    sums = jnp.dot(onehot_t, h, preferred_element_type=jnp.float32)
    counts = jnp.sum(onehot_t, axis=1, keepdims=True)
    pooled = sums / jnp.maximum(counts, 1.0)
    o_ref[...] = (
        jnp.dot(pooled, Wc_ref[...], preferred_element_type=jnp.float32)
        + bc_ref[...]
    )


def _tc_tail(xa, p0, p1, batch_row, W2, b2, Wc, bc, b1):
    return pl.pallas_call(
        _tail_body,
        out_shape=jax.ShapeDtypeStruct((N_GRAPHS, 2), jnp.float32),
    )(xa, p0, p1, batch_row, W2, b2, Wc, bc, b1)


def kernel(x, edge_index, batch, W1, b1, W2, b2, Wc, bc):
    src = edge_index[0].astype(jnp.int32)
    dst = edge_index[1].astype(jnp.int32)
    npad = PAD_E - N_EDGES
    src_p = jnp.concatenate([src, jnp.zeros((npad,), jnp.int32)])
    dst_p = jnp.concatenate([dst, jnp.full((npad,), DUMMY_ROW, jnp.int32)])
    srcm = src_p.reshape(NW * NB, BATCH_SZ)
    dstm = dst_p.reshape(NW * NB, BATCH_SZ)

    xa = _tc_xw(x, W1)
    part = _sc_scatter(xa, srcm, dstm)
    p0 = part[0, :N_NODES]
    p1 = part[1, :N_NODES]

    batch_row = batch.astype(jnp.int32).reshape(1, N_NODES)
    return _tc_tail(
        xa, p0, p1, batch_row,
        W2, b2.reshape(1, HIDDEN), Wc, bc.reshape(1, 2), b1.reshape(1, HIDDEN),
    )


# speedup vs baseline: 10.6137x; 1.2046x over previous
import functools

import jax
import jax.numpy as jnp
from jax import lax
from jax.experimental import pallas as pl
from jax.experimental.pallas import tpu as pltpu
from jax.experimental.pallas import tpu_sc as plsc

N_NODES = 10000
D_FEAT = 128
HIDDEN = 32
N_GRAPHS = 64
N_EDGES = 320000

NC = 2
NS = 16
NW = NC * NS
LANES = 16

BATCH_SZ = 128
NB = 80
EPW = NB * BATCH_SZ
PAD_E = NW * EPW
ROWS_PAD = 10112
RPS = ROWS_PAD // NS
NBUF = 4
DUMMY_ROW = N_NODES


def _xw_body(x_ref, w_ref, o_ref):
    o_ref[...] = jnp.dot(x_ref[...], w_ref[...], preferred_element_type=jnp.float32)


def _tc_xw(x, W1):
    return pl.pallas_call(
        _xw_body,
        out_shape=jax.ShapeDtypeStruct((N_NODES, HIDDEN), jnp.float32),
    )(x, W1)


def _sc_scatter(xa, srcm, dstm):
    mesh = plsc.VectorSubcoreMesh(
        core_axis_name="c", subcore_axis_name="s", num_cores=NC, num_subcores=NS
    )

    @functools.partial(
        pl.kernel,
        out_type=jax.ShapeDtypeStruct((NC, ROWS_PAD, HIDDEN), jnp.float32),
        mesh=mesh,
        scratch_types=[
            pltpu.VMEM((NB, BATCH_SZ), jnp.int32),
            pltpu.VMEM((NB, BATCH_SZ), jnp.int32),
            pltpu.VMEM((NBUF, BATCH_SZ, HIDDEN), jnp.float32),
            pltpu.VMEM((RPS, HIDDEN), jnp.float32),
            pltpu.VMEM_SHARED((ROWS_PAD, HIDDEN), jnp.float32),
            pltpu.SemaphoreType.DMA,
            pltpu.SemaphoreType.DMA,
            pltpu.SemaphoreType.DMA,
            pltpu.SemaphoreType.DMA,
        ],
        compiler_params=pltpu.CompilerParams(use_tc_tiling_on_sc=False),
    )
    def k(xa_hbm, srcm_hbm, dstm_hbm, out_hbm, srcbuf, dstbuf, rows, zbuf, aggsh,
          sem0, sem1, sem2, sem3):
        sems = (sem0, sem1, sem2, sem3)
        c = lax.axis_index("c")
        s = lax.axis_index("s")
        wid = s * NC + c

        def zrow(r, carry):
            z = jnp.zeros((LANES,), jnp.float32)
            zbuf[r, pl.ds(0, LANES)] = z
            zbuf[r, pl.ds(LANES, LANES)] = z
            return carry

        lax.fori_loop(0, RPS, zrow, 0)
        pltpu.sync_copy(zbuf, aggsh.at[pl.ds(s * RPS, RPS)])

        pltpu.sync_copy(srcm_hbm.at[pl.ds(wid * NB, NB)], srcbuf)
        pltpu.sync_copy(dstm_hbm.at[pl.ds(wid * NB, NB)], dstbuf)
        plsc.subcore_barrier()

        for b in range(NBUF):
            pltpu.async_copy(xa_hbm.at[srcbuf.at[b]], rows.at[b], sems[b])

        def group(g, carry):
            for b in range(NBUF):
                j = g * NBUF + b
                pltpu.make_async_copy(
                    xa_hbm.at[srcbuf.at[j]], rows.at[b], sems[b]
                ).wait()
                pltpu.sync_copy(rows.at[b], aggsh.at[dstbuf.at[j]], add=True)
                jn = jnp.minimum(j + NBUF, NB - 1)
                pltpu.async_copy(xa_hbm.at[srcbuf.at[jn]], rows.at[b], sems[b])
            return carry

        lax.fori_loop(0, NB // NBUF, group, 0)
        for b in range(NBUF):
            pltpu.make_async_copy(
                xa_hbm.at[srcbuf.at[0]], rows.at[b], sems[b]
            ).wait()
        plsc.subcore_barrier()

        pltpu.sync_copy(
            aggsh.at[pl.ds(s * RPS, RPS)], out_hbm.at[c, pl.ds(s * RPS, RPS)]
        )

    return k(xa, srcm, dstm)


def _tail_body(xa_ref, p0_ref, p1_ref, bt_ref, W2_ref, b2_ref, Wc_ref, bc_ref,
               b1_ref, o_ref):
    h1 = jnp.maximum(xa_ref[...] + p0_ref[...] + p1_ref[...] + b1_ref[...], 0.0)
    h = jnp.dot(h1, W2_ref[...], preferred_element_type=jnp.float32) + b2_ref[...]
    h = jnp.maximum(h, 0.0)
    gids = lax.broadcasted_iota(jnp.int32, (N_GRAPHS, N_NODES), 0)
    onehot_t = (gids == bt_ref[...]).astype(jnp.float32)
    sums = jnp.dot(onehot_t, h, preferred_element_type=jnp.float32)
    counts = jnp.sum(onehot_t, axis=1, keepdims=True)
    pooled = sums / jnp.maximum(counts, 1.0)
    o_ref[...] = (
        jnp.dot(pooled, Wc_ref[...], preferred_element_type=jnp.float32)
        + bc_ref[...]
    )


def _tc_tail(xa, p0, p1, batch_row, W2, b2, Wc, bc, b1):
    return pl.pallas_call(
        _tail_body,
        out_shape=jax.ShapeDtypeStruct((N_GRAPHS, 2), jnp.float32),
    )(xa, p0, p1, batch_row, W2, b2, Wc, bc, b1)


def kernel(x, edge_index, batch, W1, b1, W2, b2, Wc, bc):
    src = edge_index[0].astype(jnp.int32)
    dst = edge_index[1].astype(jnp.int32)
    npad = PAD_E - N_EDGES
    src_p = jnp.concatenate([src, jnp.zeros((npad,), jnp.int32)])
    dst_p = jnp.concatenate([dst, jnp.full((npad,), DUMMY_ROW, jnp.int32)])
    srcm = src_p.reshape(NW * NB, BATCH_SZ)
    dstm = dst_p.reshape(NW * NB, BATCH_SZ)

    xa = _tc_xw(x, W1)
    part = _sc_scatter(xa, srcm, dstm)
    p0 = part[0, :N_NODES]
    p1 = part[1, :N_NODES]

    batch_row = batch.astype(jnp.int32).reshape(1, N_NODES)
    return _tc_tail(
        xa, p0, p1, batch_row,
        W2, b2.reshape(1, HIDDEN), Wc, bc.reshape(1, 2), b1.reshape(1, HIDDEN),
    )


# speedup vs baseline: 10.8716x; 1.0243x over previous
import functools

import jax
import jax.numpy as jnp
from jax import lax
from jax.experimental import pallas as pl
from jax.experimental.pallas import tpu as pltpu
from jax.experimental.pallas import tpu_sc as plsc

N_NODES = 10000
D_FEAT = 128
HIDDEN = 32
N_GRAPHS = 64
N_EDGES = 320000

NC = 2
NS = 16
NW = NC * NS
LANES = 16

BATCH_SZ = 128
NB = 80
EPW = NB * BATCH_SZ
PAD_E = NW * EPW
ROWS_PAD = 10112
RPS = ROWS_PAD // NS
NBUF = 8
HDEPTH = 4
DUMMY_ROW = N_NODES


def _xw_body(x_ref, w_ref, o_ref):
    o_ref[...] = jnp.dot(x_ref[...], w_ref[...], preferred_element_type=jnp.float32)


def _tc_xw(x, W1):
    return pl.pallas_call(
        _xw_body,
        out_shape=jax.ShapeDtypeStruct((N_NODES, HIDDEN), jnp.float32),
    )(x, W1)


def _sc_scatter(xa, srcm, dstm):
    mesh = plsc.VectorSubcoreMesh(
        core_axis_name="c", subcore_axis_name="s", num_cores=NC, num_subcores=NS
    )

    @functools.partial(
        pl.kernel,
        out_type=jax.ShapeDtypeStruct((NC, ROWS_PAD, HIDDEN), jnp.float32),
        mesh=mesh,
        scratch_types=[
            pltpu.VMEM((NB, BATCH_SZ), jnp.int32),
            pltpu.VMEM((NB, BATCH_SZ), jnp.int32),
            pltpu.VMEM((NBUF, BATCH_SZ, HIDDEN), jnp.float32),
            pltpu.VMEM((RPS, HIDDEN), jnp.float32),
            pltpu.VMEM_SHARED((ROWS_PAD, HIDDEN), jnp.float32),
            pltpu.SemaphoreType.DMA((NBUF,)),
            pltpu.SemaphoreType.DMA((NBUF,)),
        ],
        compiler_params=pltpu.CompilerParams(use_tc_tiling_on_sc=False),
    )
    def k(xa_hbm, srcm_hbm, dstm_hbm, out_hbm, srcbuf, dstbuf, rows, zbuf, aggsh,
          gsem, ssem):
        c = lax.axis_index("c")
        s = lax.axis_index("s")
        wid = s * NC + c

        def zrow(r, carry):
            z = jnp.zeros((LANES,), jnp.float32)
            zbuf[r, pl.ds(0, LANES)] = z
            zbuf[r, pl.ds(LANES, LANES)] = z
            return carry

        lax.fori_loop(0, RPS, zrow, 0)
        pltpu.sync_copy(zbuf, aggsh.at[pl.ds(s * RPS, RPS)])

        pltpu.sync_copy(srcm_hbm.at[pl.ds(wid * NB, NB)], srcbuf)
        pltpu.sync_copy(dstm_hbm.at[pl.ds(wid * NB, NB)], dstbuf)
        plsc.subcore_barrier()

        def fire_gather(j, b):
            pltpu.async_copy(xa_hbm.at[srcbuf.at[j]], rows.at[b], gsem.at[b])

        def wait_gather(b):
            pltpu.make_async_copy(
                xa_hbm.at[srcbuf.at[0]], rows.at[b], gsem.at[b]
            ).wait()

        def fire_scatter(j, b):
            pltpu.async_copy(rows.at[b], aggsh.at[dstbuf.at[j]], ssem.at[b],
                             add=True)

        def wait_scatter(b):
            pltpu.make_async_copy(
                rows.at[b], aggsh.at[dstbuf.at[0]], ssem.at[b]
            ).wait()

        for b in range(HDEPTH):
            fire_gather(b, b)

        for j in range(HDEPTH):
            wait_gather(j)
            fire_scatter(j, j)
            fire_gather(j + HDEPTH, j + HDEPTH)

        def group(g, carry):
            for bi in range(NBUF):
                j = HDEPTH + g * NBUF + bi
                b = (HDEPTH + bi) % NBUF
                wait_gather(b)
                fire_scatter(j, b)
                bf = bi % NBUF
                wait_scatter(bf)
                fire_gather(j + HDEPTH, bf)
            return carry

        lax.fori_loop(0, (NB - 2 * HDEPTH) // NBUF, group, 0)

        for j in range(NB - HDEPTH, NB):
            b = j % NBUF
            wait_gather(b)
            fire_scatter(j, b)
        for b in range(NBUF):
            wait_scatter(b)
        plsc.subcore_barrier()

        pltpu.sync_copy(
            aggsh.at[pl.ds(s * RPS, RPS)], out_hbm.at[c, pl.ds(s * RPS, RPS)]
        )

    return k(xa, srcm, dstm)


def _tail_body(xa_ref, p0_ref, p1_ref, bt_ref, W2_ref, b2_ref, Wc_ref, bc_ref,
               b1_ref, o_ref):
    h1 = jnp.maximum(xa_ref[...] + p0_ref[...] + p1_ref[...] + b1_ref[...], 0.0)
    h = jnp.dot(h1, W2_ref[...], preferred_element_type=jnp.float32) + b2_ref[...]
    h = jnp.maximum(h, 0.0)
    gids = lax.broadcasted_iota(jnp.int32, (N_GRAPHS, N_NODES), 0)
    onehot_t = (gids == bt_ref[...]).astype(jnp.float32)
    sums = jnp.dot(onehot_t, h, preferred_element_type=jnp.float32)
    counts = jnp.sum(onehot_t, axis=1, keepdims=True)
    pooled = sums / jnp.maximum(counts, 1.0)
    o_ref[...] = (
        jnp.dot(pooled, Wc_ref[...], preferred_element_type=jnp.float32)
        + bc_ref[...]
    )


def _tc_tail(xa, p0, p1, batch_row, W2, b2, Wc, bc, b1):
    return pl.pallas_call(
        _tail_body,
        out_shape=jax.ShapeDtypeStruct((N_GRAPHS, 2), jnp.float32),
    )(xa, p0, p1, batch_row, W2, b2, Wc, bc, b1)


def kernel(x, edge_index, batch, W1, b1, W2, b2, Wc, bc):
    src = edge_index[0].astype(jnp.int32)
    dst = edge_index[1].astype(jnp.int32)
    npad = PAD_E - N_EDGES
    src_p = jnp.concatenate([src, jnp.zeros((npad,), jnp.int32)])
    dst_p = jnp.concatenate([dst, jnp.full((npad,), DUMMY_ROW, jnp.int32)])
    srcm = src_p.reshape(NW * NB, BATCH_SZ)
    dstm = dst_p.reshape(NW * NB, BATCH_SZ)

    xa = _tc_xw(x, W1)
    part = _sc_scatter(xa, srcm, dstm)
    p0 = part[0, :N_NODES]
    p1 = part[1, :N_NODES]

    batch_row = batch.astype(jnp.int32).reshape(1, N_NODES)
    return _tc_tail(
        xa, p0, p1, batch_row,
        W2, b2.reshape(1, HIDDEN), Wc, bc.reshape(1, 2), b1.reshape(1, HIDDEN),
    )


# speedup vs baseline: 10.8853x; 1.0013x over previous
import functools

import jax
import jax.numpy as jnp
from jax import lax
from jax.experimental import pallas as pl
from jax.experimental.pallas import tpu as pltpu
from jax.experimental.pallas import tpu_sc as plsc

N_NODES = 10000
D_FEAT = 128
HIDDEN = 32
N_GRAPHS = 64
N_EDGES = 320000

NC = 2
NS = 16
NW = NC * NS
LANES = 16

BATCH_SZ = 128
NB = 80
EPW = NB * BATCH_SZ
PAD_E = NW * EPW
ROWS_PAD = 10112
RPS = ROWS_PAD // NS
NBUF = 8
HDEPTH = 4
DUMMY_ROW = N_NODES


def _xw_body(x_ref, w_ref, o_ref):
    o_ref[...] = jnp.dot(x_ref[...], w_ref[...], preferred_element_type=jnp.float32)


def _tc_xw(x, W1):
    return pl.pallas_call(
        _xw_body,
        out_shape=jax.ShapeDtypeStruct((N_NODES, HIDDEN), jnp.float32),
    )(x, W1)


def _sc_scatter(xa, srcm, dstm):
    mesh = plsc.VectorSubcoreMesh(
        core_axis_name="c", subcore_axis_name="s", num_cores=NC, num_subcores=NS
    )

    @functools.partial(
        pl.kernel,
        out_type=jax.ShapeDtypeStruct((NC, ROWS_PAD, HIDDEN), jnp.float32),
        mesh=mesh,
        scratch_types=[
            pltpu.VMEM((NB, BATCH_SZ), jnp.int32),
            pltpu.VMEM((NB, BATCH_SZ), jnp.int32),
            pltpu.VMEM((NBUF, BATCH_SZ, HIDDEN), jnp.float32),
            pltpu.VMEM((RPS, HIDDEN), jnp.float32),
            pltpu.VMEM_SHARED((ROWS_PAD, HIDDEN), jnp.float32),
            pltpu.SemaphoreType.DMA((NBUF,)),
            pltpu.SemaphoreType.DMA((NBUF,)),
        ],
        compiler_params=pltpu.CompilerParams(use_tc_tiling_on_sc=False),
    )
    def k(xa_hbm, srcm_hbm, dstm_hbm, out_hbm, srcbuf, dstbuf, rows, zbuf, aggsh,
          gsem, ssem):
        c = lax.axis_index("c")
        s = lax.axis_index("s")
        wid = s * NC + c

        def zrow(r, carry):
            z = jnp.zeros((LANES,), jnp.float32)
            zbuf[r, pl.ds(0, LANES)] = z
            zbuf[r, pl.ds(LANES, LANES)] = z
            return carry

        lax.fori_loop(0, RPS, zrow, 0)
        pltpu.sync_copy(zbuf, aggsh.at[pl.ds(s * RPS, RPS)])

        pltpu.sync_copy(srcm_hbm.at[pl.ds(wid * NB, NB)], srcbuf)
        pltpu.sync_copy(dstm_hbm.at[pl.ds(wid * NB, NB)], dstbuf)
        plsc.subcore_barrier()

        def fire_gather(j, b):
            pltpu.async_copy(xa_hbm.at[srcbuf.at[j]], rows.at[b], gsem.at[b])

        def wait_gather(b):
            pltpu.make_async_copy(
                xa_hbm.at[srcbuf.at[0]], rows.at[b], gsem.at[b]
            ).wait()

        def fire_scatter(j, b):
            pltpu.async_copy(rows.at[b], aggsh.at[dstbuf.at[j]], ssem.at[b],
                             add=True)

        def wait_scatter(b):
            pltpu.make_async_copy(
                rows.at[b], aggsh.at[dstbuf.at[0]], ssem.at[b]
            ).wait()

        for b in range(HDEPTH):
            fire_gather(b, b)

        for j in range(HDEPTH):
            wait_gather(j)
            fire_scatter(j, j)
            fire_gather(j + HDEPTH, j + HDEPTH)

        def group(g, carry):
            for bi in range(NBUF):
                j = HDEPTH + g * NBUF + bi
                b = (HDEPTH + bi) % NBUF
                wait_gather(b)
                fire_scatter(j, b)
                bf = bi % NBUF
                wait_scatter(bf)
                fire_gather(j + HDEPTH, bf)
            return carry

        lax.fori_loop(0, (NB - 2 * HDEPTH) // NBUF, group, 0)

        for j in range(NB - HDEPTH, NB):
            b = j % NBUF
            wait_gather(b)
            fire_scatter(j, b)
        for b in range(NBUF):
            wait_scatter(b)
        plsc.subcore_barrier()

        pltpu.sync_copy(
            aggsh.at[pl.ds(s * RPS, RPS)], out_hbm.at[c, pl.ds(s * RPS, RPS)]
        )

    return k(xa, srcm, dstm)


def _tail_body(xa_ref, p0_ref, p1_ref, bt_ref, W2_ref, b2_ref, Wc_ref, bc_ref,
               b1_ref, o_ref):
    h1 = jnp.maximum(xa_ref[...] + p0_ref[...] + p1_ref[...] + b1_ref[...], 0.0)
    h = jnp.dot(h1, W2_ref[...], preferred_element_type=jnp.float32) + b2_ref[...]
    h = jnp.maximum(h, 0.0)
    gids = lax.broadcasted_iota(jnp.int32, (N_GRAPHS, N_NODES), 0)
    onehot_t = (gids == bt_ref[...]).astype(jnp.float32)
    sums = jnp.dot(onehot_t, h, preferred_element_type=jnp.float32)
    counts = jnp.sum(onehot_t, axis=1, keepdims=True)
    pooled = sums / jnp.maximum(counts, 1.0)
    o_ref[...] = (
        jnp.dot(pooled, Wc_ref[...], preferred_element_type=jnp.float32)
        + bc_ref[...]
    )


def _tc_tail(xa, p0, p1, batch_row, W2, b2, Wc, bc, b1):
    return pl.pallas_call(
        _tail_body,
        out_shape=jax.ShapeDtypeStruct((N_GRAPHS, 2), jnp.float32),
    )(xa, p0, p1, batch_row, W2, b2, Wc, bc, b1)


def kernel(x, edge_index, batch, W1, b1, W2, b2, Wc, bc):
    src = edge_index[0].astype(jnp.int32)
    dst = edge_index[1].astype(jnp.int32)
    npad = PAD_E - N_EDGES
    src_p = jnp.concatenate([src, jnp.zeros((npad,), jnp.int32)])
    pad_dst = DUMMY_ROW + jnp.arange(npad, dtype=jnp.int32) % (ROWS_PAD - DUMMY_ROW)
    dst_p = jnp.concatenate([dst, pad_dst])
    srcm = src_p.reshape(NW * NB, BATCH_SZ)
    dstm = dst_p.reshape(NW * NB, BATCH_SZ)

    xa = _tc_xw(x, W1)
    part = _sc_scatter(xa, srcm, dstm)
    p0 = part[0, :N_NODES]
    p1 = part[1, :N_NODES]

    batch_row = batch.astype(jnp.int32).reshape(1, N_NODES)
    return _tc_tail(
        xa, p0, p1, batch_row,
        W2, b2.reshape(1, HIDDEN), Wc, bc.reshape(1, 2), b1.reshape(1, HIDDEN),
    )
